# Initial kernel scaffold; baseline (speedup 1.0000x reference)
import jax
import jax.numpy as jnp
from jax.experimental import pallas as pl


def kernel(x, adj, W, b):
    raise NotImplementedError("write your pallas kernel here")


# speedup vs baseline: 1.0054x; 1.0054x over previous
import jax
import jax.numpy as jnp
from jax.experimental import pallas as pl
from jax.experimental.pallas import tpu as pltpu

_N = 10000
_D = 128
_BM = 400
_NI = _N // _BM


def _gcn_kernel(x_ref, w_ref, b_ref, adj_ref, out_ref, supp_ref, sum_ref):
    i = pl.program_id(0)

    @pl.when(i == 0)
    def _init():
        supp_ref[...] = jnp.dot(x_ref[...], w_ref[...],
                                preferred_element_type=jnp.float32)
        sum_ref[...] = jnp.zeros_like(sum_ref)

    h = jnp.tanh(jnp.dot(adj_ref[...], supp_ref[...],
                         preferred_element_type=jnp.float32) + b_ref[...])
    sum_ref[...] += jnp.sum(h, axis=0, keepdims=True)

    @pl.when(i == _NI - 1)
    def _write_out():
        out_ref[...] = sum_ref[...] * (1.0 / _N)


def kernel(x, adj, W, b):
    b2 = b.reshape(1, _D)
    out = pl.pallas_call(
        _gcn_kernel,
        grid=(_NI,),
        in_specs=[
            pl.BlockSpec((_N, _D), lambda i: (0, 0)),
            pl.BlockSpec((_D, _D), lambda i: (0, 0)),
            pl.BlockSpec((1, _D), lambda i: (0, 0)),
            pl.BlockSpec((_BM, _N), lambda i: (i, 0)),
        ],
        out_specs=pl.BlockSpec((1, _D), lambda i: (0, 0)),
        out_shape=jax.ShapeDtypeStruct((1, _D), jnp.float32),
        scratch_shapes=[
            pltpu.VMEM((_N, _D), jnp.float32),
            pltpu.VMEM((1, _D), jnp.float32),
        ],
    )(x, W, b2, adj)
    return out.reshape(_D)


# speedup vs baseline: 1.0190x; 1.0136x over previous
import jax
import jax.numpy as jnp
from jax.experimental import pallas as pl
from jax.experimental.pallas import tpu as pltpu

_N = 10000
_D = 128
_BM = 400
_NI = _N // _BM


def _gcn_kernel(x_ref, w_ref, b_ref, adj_ref, out_ref, supp_ref, sum_ref):
    i = pl.program_id(0)

    @pl.when(i == 0)
    def _init():
        supp_ref[...] = jnp.dot(x_ref[...], w_ref[...],
                                preferred_element_type=jnp.float32
                                ).astype(jnp.bfloat16)
        sum_ref[...] = jnp.zeros_like(sum_ref)

    h = jnp.tanh(jnp.dot(adj_ref[...].astype(jnp.bfloat16), supp_ref[...],
                         preferred_element_type=jnp.float32) + b_ref[...])
    sum_ref[...] += jnp.sum(h, axis=0, keepdims=True)

    @pl.when(i == _NI - 1)
    def _write_out():
        out_ref[...] = sum_ref[...] * (1.0 / _N)


def kernel(x, adj, W, b):
    b2 = b.reshape(1, _D)
    out = pl.pallas_call(
        _gcn_kernel,
        grid=(_NI,),
        in_specs=[
            pl.BlockSpec((_N, _D), lambda i: (0, 0)),
            pl.BlockSpec((_D, _D), lambda i: (0, 0)),
            pl.BlockSpec((1, _D), lambda i: (0, 0)),
            pl.BlockSpec((_BM, _N), lambda i: (i, 0)),
        ],
        out_specs=pl.BlockSpec((1, _D), lambda i: (0, 0)),
        out_shape=jax.ShapeDtypeStruct((1, _D), jnp.float32),
        scratch_shapes=[
            pltpu.VMEM((_N, _D), jnp.bfloat16),
            pltpu.VMEM((1, _D), jnp.float32),
        ],
    )(x, W, b2, adj)
    return out.reshape(_D)
